# baseline (speedup 1.0000x reference)
import functools

import jax
import jax.numpy as jnp
from jax import lax
from jax.experimental import pallas as pl
from jax.experimental.pallas import tpu as pltpu
from jax.experimental.pallas import tpu_sc as plsc

BATCH = 16384
EMBED_DIM = 32
NUM_WORKERS = 32
B_PER_W = BATCH // NUM_WORKERS
CHUNK = 128
NCHUNK = B_PER_W // CHUNK
GROUPS = B_PER_W // 16


def _gmf_body(uidx_hbm, iidx_hbm, utab_hbm, itab_hbm, wb_hbm, bvec_hbm,
              out_hbm, uidx_v, iidx_v, urows_v, irows_v, wb_v, bv_v,
              out_v, usem, isem):
  wid = lax.axis_index("s") * 2 + lax.axis_index("c")
  base = wid * B_PER_W

  pltpu.sync_copy(uidx_hbm.at[wid], uidx_v)
  pltpu.sync_copy(iidx_hbm.at[wid], iidx_v)
  pltpu.sync_copy(wb_hbm, wb_v)
  pltpu.sync_copy(bvec_hbm, bv_v)

  copies = []
  for k in range(NCHUNK):
    sl = pl.ds(k * CHUNK, CHUNK)
    copies.append(
        pltpu.async_copy(utab_hbm.at[uidx_v.at[k]], urows_v.at[sl], usem))
    copies.append(
        pltpu.async_copy(itab_hbm.at[iidx_v.at[k]], irows_v.at[sl], isem))
  for c in copies:
    c.wait()

  bv = bv_v[...]
  lane = lax.iota(jnp.int32, 16)

  def group(g, carry):
    rows = lane + g * 16
    acc = jnp.zeros((16,), jnp.float32)
    for d in range(EMBED_DIM):
      col = jnp.full((16,), d, jnp.int32)
      u = plsc.load_gather(urows_v, [rows, col])
      v = plsc.load_gather(irows_v, [rows, col])
      acc = acc + u * v * wb_v[d, :]
    z = acc + bv
    res = 5.0 / (1.0 + jnp.exp(-z)) + 1.0
    out_v[pl.ds(g * 16, 16)] = res
    return carry

  lax.fori_loop(0, GROUPS, group, 0)

  pltpu.sync_copy(out_v, out_hbm.at[pl.ds(base, B_PER_W)])


@jax.jit
def kernel(user_indices, item_indices, user_table, item_table, W, b):
  uidx3 = user_indices.astype(jnp.int32).reshape(NUM_WORKERS, NCHUNK, CHUNK)
  iidx3 = item_indices.astype(jnp.int32).reshape(NUM_WORKERS, NCHUNK, CHUNK)
  wb = jnp.broadcast_to(W, (EMBED_DIM, 16))
  bvec = jnp.broadcast_to(b, (16,))

  mesh = plsc.VectorSubcoreMesh(core_axis_name="c", subcore_axis_name="s")
  run = functools.partial(
      pl.kernel,
      mesh=mesh,
      compiler_params=pltpu.CompilerParams(
          use_tc_tiling_on_sc=False, needs_layout_passes=False),
      out_type=jax.ShapeDtypeStruct((BATCH,), jnp.float32),
      scratch_types=[
          pltpu.VMEM((NCHUNK, CHUNK), jnp.int32),
          pltpu.VMEM((NCHUNK, CHUNK), jnp.int32),
          pltpu.VMEM((B_PER_W, EMBED_DIM), jnp.float32),
          pltpu.VMEM((B_PER_W, EMBED_DIM), jnp.float32),
          pltpu.VMEM((EMBED_DIM, 16), jnp.float32),
          pltpu.VMEM((16,), jnp.float32),
          pltpu.VMEM((B_PER_W,), jnp.float32),
          pltpu.SemaphoreType.DMA,
          pltpu.SemaphoreType.DMA,
      ],
  )(_gmf_body)
  out = run(uidx3, iidx3, user_table, item_table, wb, bvec)
  return out.reshape(BATCH, 1)


# speedup vs baseline: 1.4870x; 1.4870x over previous
import functools

import jax
import jax.numpy as jnp
from jax import lax
from jax.experimental import pallas as pl
from jax.experimental.pallas import tpu as pltpu
from jax.experimental.pallas import tpu_sc as plsc

BATCH = 16384
EMBED_DIM = 32
NUM_WORKERS = 32
B_PER_W = BATCH // NUM_WORKERS
CHUNK = 128
NCHUNK = B_PER_W // CHUNK
GROUPS_PER_CHUNK = CHUNK // 16


def _fire_rows(tab_hbm, idx_v, buf, sem, idx_base):
  def fire(j, carry):
    r = idx_v[pl.ds(idx_base + j, 16)][0]
    pltpu.make_async_copy(tab_hbm.at[r], buf.at[j], sem).start()
    return carry
  lax.fori_loop(0, CHUNK, fire, 0)


def _drain_rows(tab_hbm, buf, sem):
  def drain(j, carry):
    pltpu.make_async_copy(tab_hbm.at[0], buf.at[0], sem).wait()
    return carry
  lax.fori_loop(0, CHUNK, drain, 0)


def _gmf_body(uidx_hbm, iidx_hbm, utab_hbm, itab_hbm, w_hbm, bvec_hbm,
              out_hbm, uidx_v, iidx_v, ubuf0, ubuf1, ibuf0, ibuf1,
              w_v, bv_v, out_v, usem0, usem1, isem0, isem1):
  wid = lax.axis_index("s") * 2 + lax.axis_index("c")
  base = wid * B_PER_W

  pltpu.sync_copy(uidx_hbm.at[pl.ds(base, B_PER_W)],
                  uidx_v.at[pl.ds(0, B_PER_W)])
  pltpu.sync_copy(iidx_hbm.at[pl.ds(base, B_PER_W)],
                  iidx_v.at[pl.ds(0, B_PER_W)])
  pltpu.sync_copy(w_hbm, w_v)
  pltpu.sync_copy(bvec_hbm, bv_v)

  bv = bv_v[...]
  w0 = w_v[pl.ds(0, 16)]
  w1 = w_v[pl.ds(16, 16)]
  lane = lax.iota(jnp.int32, 16)
  ubufs = (ubuf0, ubuf1)
  ibufs = (ibuf0, ibuf1)
  usems = (usem0, usem1)
  isems = (isem0, isem1)

  _fire_rows(utab_hbm, uidx_v, ubufs[0], usems[0], 0)
  _fire_rows(itab_hbm, iidx_v, ibufs[0], isems[0], 0)

  for c in range(NCHUNK):
    cb = c % 2
    nb = (c + 1) % 2
    if c + 1 < NCHUNK:
      _fire_rows(utab_hbm, uidx_v, ubufs[nb], usems[nb], (c + 1) * CHUNK)
      _fire_rows(itab_hbm, iidx_v, ibufs[nb], isems[nb], (c + 1) * CHUNK)
    _drain_rows(utab_hbm, ubufs[cb], usems[cb])
    _drain_rows(itab_hbm, ibufs[cb], isems[cb])

    ubuf = ubufs[cb]
    ibuf = ibufs[cb]

    def group(g, carry, ubuf=ubuf, ibuf=ibuf, c=c):
      rows = lane + g * 16
      acc = jnp.zeros((16,), jnp.float32)
      for d in range(EMBED_DIM):
        col = jnp.full((16,), d, jnp.int32)
        u = plsc.load_gather(ubuf, [rows, col])
        v = plsc.load_gather(ibuf, [rows, col])
        w_s = (w0 if d < 16 else w1)[d % 16]
        acc = acc + u * v * jnp.full((16,), w_s, jnp.float32)
      z = acc + bv
      res = 5.0 / (1.0 + jnp.exp(-z)) + 1.0
      out_v[pl.ds(c * CHUNK + g * 16, 16)] = res
      return carry

    lax.fori_loop(0, GROUPS_PER_CHUNK, group, 0)

  pltpu.sync_copy(out_v, out_hbm.at[pl.ds(base, B_PER_W)])


@jax.jit
def kernel(user_indices, item_indices, user_table, item_table, W, b):
  uidx = user_indices.astype(jnp.int32)
  iidx = item_indices.astype(jnp.int32)
  wvec = W.reshape(EMBED_DIM)
  bvec = jnp.broadcast_to(b, (16,))

  mesh = plsc.VectorSubcoreMesh(core_axis_name="c", subcore_axis_name="s")
  run = functools.partial(
      pl.kernel,
      mesh=mesh,
      compiler_params=pltpu.CompilerParams(needs_layout_passes=False),
      out_type=jax.ShapeDtypeStruct((BATCH,), jnp.float32),
      scratch_types=[
          pltpu.VMEM((B_PER_W + 16,), jnp.int32),
          pltpu.VMEM((B_PER_W + 16,), jnp.int32),
          pltpu.VMEM((CHUNK, EMBED_DIM), jnp.float32),
          pltpu.VMEM((CHUNK, EMBED_DIM), jnp.float32),
          pltpu.VMEM((CHUNK, EMBED_DIM), jnp.float32),
          pltpu.VMEM((CHUNK, EMBED_DIM), jnp.float32),
          pltpu.VMEM((EMBED_DIM,), jnp.float32),
          pltpu.VMEM((16,), jnp.float32),
          pltpu.VMEM((B_PER_W,), jnp.float32),
          pltpu.SemaphoreType.DMA,
          pltpu.SemaphoreType.DMA,
          pltpu.SemaphoreType.DMA,
          pltpu.SemaphoreType.DMA,
      ],
  )(_gmf_body)
  out = run(uidx, iidx, user_table, item_table, wvec, bvec)
  return out.reshape(BATCH, 1)
